# baseline (speedup 1.0000x reference)
import functools

import jax
import jax.numpy as jnp
from jax import lax
from jax.experimental import pallas as pl
from jax.experimental.pallas import tpu as pltpu
from jax.experimental.pallas import tpu_sc as plsc


@functools.lru_cache(maxsize=None)
def _build(V: int, D: int, B: int, H: int):
    info = plsc.get_sparse_core_info()
    NC, NS, L = info.num_cores, info.num_subcores, info.num_lanes
    NW = NC * NS
    assert D == 32 and L == 16 and H == 50 and V == 1000000
    assert B % NW == 0 and (B // NW) % (8 * L) == 0
    b_per_w = B // NW
    WROWS = V // 4
    NBLK = V // 128
    TAIL = V - NBLK * 128
    base_cnt = NBLK // NW
    extra = NBLK - base_cnt * NW

    mesh = plsc.VectorSubcoreMesh(core_axis_name="c", subcore_axis_name="s")
    cparams = pltpu.CompilerParams(
        use_tc_tiling_on_sc=True, needs_layout_passes=False)


    @functools.partial(
        pl.kernel,
        out_type=jax.ShapeDtypeStruct((WROWS, 128), jnp.float32),
        mesh=mesh,
        compiler_params=cparams,
        scratch_types=[
            pltpu.VMEM((2, D, 128), jnp.float32),
            pltpu.VMEM((2, D, 128), jnp.float32),
            pltpu.VMEM((D, TAIL), jnp.float32),
            pltpu.VMEM((TAIL // 4, 128), jnp.float32),
            pltpu.SemaphoreType.DMA,
            pltpu.SemaphoreType.DMA,
            pltpu.SemaphoreType.DMA,
            pltpu.SemaphoreType.DMA,
        ],
    )
    def transpose_kernel(wt_hbm, w_hbm, inb, outb, intail, outtail,
                         isem0, isem1, osem0, osem1):
        wid = lax.axis_index("s") * NC + lax.axis_index("c")
        isems = (isem0, isem1)
        osems = (osem0, osem1)
        lane = jnp.arange(L, dtype=jnp.int32)
        dv = (lane, L + lane)

        def in_cp(buf, g):
            return pltpu.make_async_copy(
                wt_hbm.at[:, pl.ds(g * 128, 128)], inb.at[buf], isems[buf])

        def out_cp(buf, g):
            return pltpu.make_async_copy(
                outb.at[buf], w_hbm.at[pl.ds(g * 32, 32), :], osems[buf])

        def transpose_block(buf):
            for w in range(32):
                vecs = [
                    plsc.load_gather(
                        inb.at[buf],
                        [dv[v % 2], jnp.zeros((L,), jnp.int32) + (4 * w + v // 2)])
                    for v in range(8)
                ]
                for v, vec in enumerate(vecs):
                    outb[buf, w, pl.ds(16 * v, 16)] = vec

        in_cp(0, wid).start()
        in_cp(1, 32 + wid).start()

        def body(j, carry):
            for sub in range(2):
                i = 2 * j + sub
                g = i * NW + wid
                in_cp(sub, g).wait()

                @pl.when(j > 0)
                def _():
                    out_cp(sub, g).wait()

                transpose_block(sub)
                out_cp(sub, g).start()
                inxt = i + 2
                ok = jnp.logical_or(
                    inxt < base_cnt,
                    jnp.logical_and(inxt == base_cnt, wid < extra))

                @pl.when(ok)
                def _():
                    in_cp(sub, inxt * NW + wid).start()
            return carry

        lax.fori_loop(0, base_cnt // 2, body, 0)

        @pl.when(wid < extra)
        def _():
            g = base_cnt * NW + wid
            in_cp(0, g).wait()
            out_cp(0, g).wait()
            transpose_block(0)
            out_cp(0, g).start()
            out_cp(0, g).wait()

        @pl.when(wid == extra)
        def _():
            pltpu.sync_copy(wt_hbm.at[:, pl.ds(NBLK * 128, TAIL)], intail)
            for w in range(TAIL // 4):
                vecs = [
                    plsc.load_gather(
                        intail,
                        [dv[v % 2], jnp.zeros((L,), jnp.int32) + (4 * w + v // 2)])
                    for v in range(8)
                ]
                for v, vec in enumerate(vecs):
                    outtail[w, pl.ds(16 * v, 16)] = vec
            pltpu.sync_copy(outtail, w_hbm.at[pl.ds(NBLK * 32, TAIL // 4), :])

        @pl.when(wid >= extra)
        def _():
            out_cp(0, 0).wait()

        out_cp(1, 0).wait()


    @functools.partial(
        pl.kernel,
        out_type=jax.ShapeDtypeStruct((H, D // 8, NW, 8, L * 8), jnp.float32),
        mesh=mesh,
        compiler_params=cparams,
        scratch_types=[
            pltpu.VMEM((b_per_w * H,), jnp.int32),
            pltpu.VMEM((b_per_w,), jnp.int32),
            pltpu.VMEM((b_per_w,), jnp.int32),
            pltpu.VMEM((b_per_w,), jnp.int32),
            pltpu.VMEM((b_per_w,), jnp.int32),
            pltpu.VMEM((2, b_per_w, 128), jnp.float32),
            pltpu.VMEM((2, D // 8, 8, b_per_w), jnp.float32),
            pltpu.SemaphoreType.DMA,
            pltpu.SemaphoreType.DMA,
            pltpu.SemaphoreType.DMA,
            pltpu.SemaphoreType.DMA,
        ],
    )
    def gather_kernel(w_hbm, idx_hbm, out_hbm, idx_v, rowid0, rowid1,
                      colb0, colb1, rows_v, stage_v,
                      gsem0, gsem1, wsem0, wsem1):
        wid = lax.axis_index("s") * NC + lax.axis_index("c")
        pltpu.sync_copy(idx_hbm.at[pl.ds(wid * b_per_w * H, b_per_w * H)], idx_v)
        rowids = (rowid0, rowid1)
        colbs = (colb0, colb1)
        gsems = (gsem0, gsem1)
        wsems = (wsem0, wsem1)
        lane = jnp.arange(L, dtype=jnp.int32)
        lane50 = lane * H
        rowvs = [jnp.zeros((L,), jnp.int32) + (blq * L + lane)
                 for blq in range(b_per_w // L)]

        def prep(h, buf):
            for blq in range(b_per_w // L):
                vidx = plsc.load_gather(idx_v, [lane50 + (blq * L * H + h)])
                rowids[buf][pl.ds(blq * L, L)] = lax.shift_right_logical(vidx, 2)
                colbs[buf][pl.ds(blq * L, L)] = lax.shift_left(
                    jnp.bitwise_and(vidx, 3), 5)

        def g_cp(buf):
            return pltpu.make_async_copy(
                w_hbm.at[rowids[buf]], rows_v.at[buf], gsems[buf])

        def w_cp(buf, h):
            return pltpu.make_async_copy(
                stage_v.at[buf], out_hbm.at[h, :, wid, :, :], wsems[buf])

        def transpose_chunk(buf):
            cbs = [colbs[buf][pl.ds(blq * L, L)] for blq in range(b_per_w // L)]
            for d in range(D):
                vecs = [
                    plsc.load_gather(rows_v.at[buf], [rowvs[blq], cbs[blq] + d])
                    for blq in range(b_per_w // L)
                ]
                for blq, vec in enumerate(vecs):
                    stage_v[buf, d // 8, d % 8, pl.ds(blq * L, L)] = vec

        prep(0, 0)
        g_cp(0).start()
        prep(1, 1)
        g_cp(1).start()

        def body(j, carry):
            for sub in range(2):
                h = 2 * j + sub
                g_cp(sub).wait()

                @pl.when(j > 0)
                def _():
                    w_cp(sub, h).wait()

                transpose_chunk(sub)
                w_cp(sub, h).start()

                @pl.when(h + 2 < H)
                def _():
                    prep(h + 2, sub)
                    g_cp(sub).start()
            return carry

        lax.fori_loop(0, H // 2, body, 0)
        w_cp(0, 0).wait()
        w_cp(1, 0).wait()

    return transpose_kernel, gather_kernel


def kernel(word_embedding, input_token_ids):
    V, D = word_embedding.shape
    B, H = input_token_ids.shape
    tk, gk = _build(V, D, B, H)
    wrm = tk(word_embedding.T)
    idx = input_token_ids.astype(jnp.int32).reshape(-1)
    out5 = gk(wrm, idx)
    return out5.transpose(2, 4, 0, 1, 3).reshape(B, H, D)


# speedup vs baseline: 1.2897x; 1.2897x over previous
import functools

import jax
import jax.numpy as jnp
from jax import lax
from jax.experimental import pallas as pl
from jax.experimental.pallas import tpu as pltpu
from jax.experimental.pallas import tpu_sc as plsc


@functools.lru_cache(maxsize=None)
def _build(V: int, D: int, B: int, H: int):
    info = plsc.get_sparse_core_info()
    NC, NS, L = info.num_cores, info.num_subcores, info.num_lanes
    NW = NC * NS
    assert D == 32 and L == 16 and H == 50 and V == 1000000
    assert B % NW == 0 and (B // NW) % (8 * L) == 0
    b_per_w = B // NW
    WROWS = V // 4
    NBLK = V // 128
    TAIL = V - NBLK * 128
    base_cnt = NBLK // NW
    extra = NBLK - base_cnt * NW

    mesh = plsc.VectorSubcoreMesh(core_axis_name="c", subcore_axis_name="s")
    cparams = pltpu.CompilerParams(
        use_tc_tiling_on_sc=True, needs_layout_passes=False)


    @functools.partial(
        pl.kernel,
        out_type=jax.ShapeDtypeStruct((WROWS, 128), jnp.float32),
        mesh=mesh,
        compiler_params=cparams,
        scratch_types=[
            pltpu.VMEM((2, D, 128), jnp.float32),
            pltpu.VMEM((2, D, 128), jnp.float32),
            pltpu.VMEM((D, TAIL), jnp.float32),
            pltpu.VMEM((TAIL // 4, 128), jnp.float32),
            pltpu.SemaphoreType.DMA,
            pltpu.SemaphoreType.DMA,
            pltpu.SemaphoreType.DMA,
            pltpu.SemaphoreType.DMA,
        ],
    )
    def transpose_kernel(wt_hbm, w_hbm, inb, outb, intail, outtail,
                         isem0, isem1, osem0, osem1):
        wid = lax.axis_index("s") * NC + lax.axis_index("c")
        isems = (isem0, isem1)
        osems = (osem0, osem1)
        lane = jnp.arange(L, dtype=jnp.int32)
        dv = (lane, L + lane)

        def in_cp(buf, g):
            return pltpu.make_async_copy(
                wt_hbm.at[:, pl.ds(g * 128, 128)], inb.at[buf], isems[buf])

        def out_cp(buf, g):
            return pltpu.make_async_copy(
                outb.at[buf], w_hbm.at[pl.ds(g * 32, 32), :], osems[buf])

        def transpose_block(buf):
            @plsc.parallel_loop(0, 32, 1, unroll=4)
            def _(w):
                for v in range(8):
                    vec = plsc.load_gather(
                        inb.at[buf],
                        [dv[v % 2], jnp.zeros((L,), jnp.int32) + (4 * w + v // 2)])
                    outb[buf, w, pl.ds(16 * v, 16)] = vec

        in_cp(0, wid).start()
        in_cp(1, 32 + wid).start()

        def body(j, carry):
            for sub in range(2):
                i = 2 * j + sub
                g = i * NW + wid
                in_cp(sub, g).wait()

                @pl.when(j > 0)
                def _():
                    out_cp(sub, g).wait()

                transpose_block(sub)
                out_cp(sub, g).start()
                inxt = i + 2
                ok = jnp.logical_or(
                    inxt < base_cnt,
                    jnp.logical_and(inxt == base_cnt, wid < extra))

                @pl.when(ok)
                def _():
                    in_cp(sub, inxt * NW + wid).start()
            return carry

        lax.fori_loop(0, base_cnt // 2, body, 0)

        @pl.when(wid < extra)
        def _():
            g = base_cnt * NW + wid
            in_cp(0, g).wait()
            out_cp(0, g).wait()
            transpose_block(0)
            out_cp(0, g).start()
            out_cp(0, g).wait()

        @pl.when(wid == extra)
        def _():
            pltpu.sync_copy(wt_hbm.at[:, pl.ds(NBLK * 128, TAIL)], intail)

            @plsc.parallel_loop(0, TAIL // 4, 1, unroll=4)
            def _(w):
                for v in range(8):
                    vec = plsc.load_gather(
                        intail,
                        [dv[v % 2], jnp.zeros((L,), jnp.int32) + (4 * w + v // 2)])
                    outtail[w, pl.ds(16 * v, 16)] = vec

            pltpu.sync_copy(outtail, w_hbm.at[pl.ds(NBLK * 32, TAIL // 4), :])

        @pl.when(wid >= extra)
        def _():
            out_cp(0, 0).wait()

        out_cp(1, 0).wait()


    @functools.partial(
        pl.kernel,
        out_type=jax.ShapeDtypeStruct((H, D // 8, NW, 8, L * 8), jnp.float32),
        mesh=mesh,
        compiler_params=cparams,
        scratch_types=[
            pltpu.VMEM((b_per_w * H,), jnp.int32),
            pltpu.VMEM((b_per_w,), jnp.int32),
            pltpu.VMEM((b_per_w,), jnp.int32),
            pltpu.VMEM((b_per_w,), jnp.int32),
            pltpu.VMEM((b_per_w,), jnp.int32),
            pltpu.VMEM((2, b_per_w, 128), jnp.float32),
            pltpu.VMEM((2, D // 8, 8, b_per_w), jnp.float32),
            pltpu.SemaphoreType.DMA,
            pltpu.SemaphoreType.DMA,
            pltpu.SemaphoreType.DMA,
            pltpu.SemaphoreType.DMA,
        ],
    )
    def gather_kernel(w_hbm, idx_hbm, out_hbm, idx_v, rowid0, rowid1,
                      colb0, colb1, rows_v, stage_v,
                      gsem0, gsem1, wsem0, wsem1):
        wid = lax.axis_index("s") * NC + lax.axis_index("c")
        pltpu.sync_copy(idx_hbm.at[pl.ds(wid * b_per_w * H, b_per_w * H)], idx_v)
        rowids = (rowid0, rowid1)
        colbs = (colb0, colb1)
        gsems = (gsem0, gsem1)
        wsems = (wsem0, wsem1)
        lane = jnp.arange(L, dtype=jnp.int32)
        lane50 = lane * H
        rowvs = [jnp.zeros((L,), jnp.int32) + (blq * L + lane)
                 for blq in range(b_per_w // L)]

        def prep(h, buf):
            for blq in range(b_per_w // L):
                vidx = plsc.load_gather(idx_v, [lane50 + (blq * L * H + h)])
                rowids[buf][pl.ds(blq * L, L)] = lax.shift_right_logical(vidx, 2)
                colbs[buf][pl.ds(blq * L, L)] = lax.shift_left(
                    jnp.bitwise_and(vidx, 3), 5)

        def g_cp(buf):
            return pltpu.make_async_copy(
                w_hbm.at[rowids[buf]], rows_v.at[buf], gsems[buf])

        def w_cp(buf, h):
            return pltpu.make_async_copy(
                stage_v.at[buf], out_hbm.at[h, :, wid, :, :], wsems[buf])

        def transpose_chunk(buf):
            cbs = [colbs[buf][pl.ds(blq * L, L)] for blq in range(b_per_w // L)]
            for r in range(D // 8):

                @plsc.parallel_loop(0, 8, 1, unroll=4)
                def _(s):
                    d = r * 8 + s
                    for blq in range(b_per_w // L):
                        vec = plsc.load_gather(
                            rows_v.at[buf], [rowvs[blq], cbs[blq] + d])
                        stage_v[buf, r, s, pl.ds(blq * L, L)] = vec

        prep(0, 0)
        g_cp(0).start()
        prep(1, 1)
        g_cp(1).start()

        def body(j, carry):
            for sub in range(2):
                h = 2 * j + sub
                g_cp(sub).wait()

                @pl.when(j > 0)
                def _():
                    w_cp(sub, h).wait()

                transpose_chunk(sub)
                w_cp(sub, h).start()

                @pl.when(h + 2 < H)
                def _():
                    prep(h + 2, sub)
                    g_cp(sub).start()
            return carry

        lax.fori_loop(0, H // 2, body, 0)
        w_cp(0, 0).wait()
        w_cp(1, 0).wait()

    return transpose_kernel, gather_kernel


def kernel(word_embedding, input_token_ids):
    V, D = word_embedding.shape
    B, H = input_token_ids.shape
    tk, gk = _build(V, D, B, H)
    wrm = tk(word_embedding.T)
    idx = input_token_ids.astype(jnp.int32).reshape(-1)
    out5 = gk(wrm, idx)
    return out5.transpose(2, 4, 0, 1, 3).reshape(B, H, D)
